# Initial kernel scaffold; baseline (speedup 1.0000x reference)
import jax
import jax.numpy as jnp
from jax.experimental import pallas as pl


def kernel(tensor1_values, tensor1_segment_ids, tensor2_values):
    raise NotImplementedError("write your pallas kernel here")


# speedup vs baseline: 5.6851x; 5.6851x over previous
import functools

import jax
import jax.numpy as jnp
from jax import lax
from jax.experimental import pallas as pl
from jax.experimental.pallas import tpu as pltpu
from jax.experimental.pallas import tpu_sc as plsc

N = 320000
D = 128
M = 128
S = 10000

NB = N // 128
NW = 32
MAXC = 80
NBP = NW * MAXC
SP = 10240
STRIPE = SP // 16
CHUNK = 128


def _sc_segment_sum(x3, ids2, zeros):
    mesh = plsc.VectorSubcoreMesh(core_axis_name="c", subcore_axis_name="s")

    @functools.partial(
        pl.kernel,
        mesh=mesh,
        out_type=jax.ShapeDtypeStruct((2, SP, D), jnp.float32),
        scratch_types=[
            pltpu.VMEM_SHARED((SP, D), jnp.float32),
            pltpu.VMEM((MAXC, 128), jnp.int32),
            pltpu.VMEM((128, D), jnp.float32),
        ],
    )
    def segsum(x_hbm, ids_hbm, z_hbm, out_hbm, acc, idx_v, rows_v):
        cid = lax.axis_index("c")
        sid = lax.axis_index("s")
        wid = sid * 2 + cid

        pltpu.sync_copy(z_hbm, rows_v)
        for k in range(STRIPE // CHUNK):
            pltpu.sync_copy(rows_v, acc.at[pl.ds(sid * STRIPE + k * CHUNK, CHUNK)])
        plsc.subcore_barrier()

        c0 = wid * MAXC
        cnt = jnp.clip(NB - c0, 0, MAXC)
        pltpu.sync_copy(ids_hbm.at[pl.ds(c0, MAXC)], idx_v)

        def body(b, carry):
            pltpu.sync_copy(x_hbm.at[c0 + b], rows_v)
            pltpu.sync_copy(rows_v, acc.at[idx_v.at[b]], add=True)
            return carry

        lax.fori_loop(0, cnt, body, 0)
        plsc.subcore_barrier()

        for k in range(STRIPE // CHUNK):
            base = sid * STRIPE + k * CHUNK
            pltpu.sync_copy(acc.at[pl.ds(base, CHUNK)], rows_v)
            pltpu.sync_copy(rows_v, out_hbm.at[cid, pl.ds(base, CHUNK)])

    return segsum(x3, ids2, zeros)


def _tc_matmul(partials, w):
    BS = 1000

    def mm(p_ref, w_ref, o_ref):
        p = p_ref[0] + p_ref[1]
        o_ref[...] = lax.dot_general(
            p, w_ref[...], (((1,), (1,)), ((), ())),
            preferred_element_type=jnp.float32,
        )

    return pl.pallas_call(
        mm,
        grid=(S // BS,),
        in_specs=[
            pl.BlockSpec((2, BS, D), lambda i: (0, i, 0)),
            pl.BlockSpec((M, D), lambda i: (0, 0)),
        ],
        out_specs=pl.BlockSpec((BS, M), lambda i: (i, 0)),
        out_shape=jax.ShapeDtypeStruct((S, M), jnp.float32),
    )(partials, w)


def kernel(tensor1_values, tensor1_segment_ids, tensor2_values):
    x3 = tensor1_values.reshape(NB, 128, D)
    ids2 = tensor1_segment_ids.astype(jnp.int32).reshape(NB, 128)
    ids2 = jnp.concatenate([ids2, jnp.zeros((NBP - NB, 128), jnp.int32)])
    zeros = jnp.zeros((128, D), jnp.float32)
    partials = _sc_segment_sum(x3, ids2, zeros)
    return _tc_matmul(partials, tensor2_values)


# speedup vs baseline: 7.7990x; 1.3718x over previous
import functools

import jax
import jax.numpy as jnp
from jax import lax
from jax.experimental import pallas as pl
from jax.experimental.pallas import tpu as pltpu
from jax.experimental.pallas import tpu_sc as plsc

N = 320000
D = 128
M = 128
S = 10000

NB = N // 128
NW = 32
MAXC = 80
NBP = NW * MAXC
SP = 10240
STRIPE = SP // 16
CHUNK = 128


def _sc_segment_sum(x3, ids2, zeros):
    mesh = plsc.VectorSubcoreMesh(core_axis_name="c", subcore_axis_name="s")

    NBUF = 2
    IDXC = 16

    @functools.partial(
        pl.kernel,
        mesh=mesh,
        out_type=jax.ShapeDtypeStruct((2, SP, D), jnp.float32),
        scratch_types=[
            pltpu.VMEM_SHARED((SP, D), jnp.float32),
            pltpu.VMEM((IDXC, 128), jnp.int32),
        ]
        + [pltpu.VMEM((128, D), jnp.float32) for _ in range(NBUF)]
        + [pltpu.SemaphoreType.DMA for _ in range(NBUF)],
    )
    def segsum(x_hbm, ids_hbm, z_hbm, out_hbm, acc, idx_v, *bufs_sems):
        bufs = bufs_sems[:NBUF]
        sems = bufs_sems[NBUF:]
        cid = lax.axis_index("c")
        sid = lax.axis_index("s")
        wid = sid * 2 + cid

        pltpu.sync_copy(z_hbm, bufs[0])
        for k in range(STRIPE // CHUNK):
            pltpu.sync_copy(bufs[0], acc.at[pl.ds(sid * STRIPE + k * CHUNK, CHUNK)])
        plsc.subcore_barrier()

        c0 = wid * MAXC
        cnt = jnp.clip(NB - c0, 0, MAXC)

        def chunk_body(h, carry):
            base = h * IDXC
            cnt_h = jnp.clip(cnt - base, 0, IDXC)

            @pl.when(cnt_h > 0)
            def _():
                pltpu.sync_copy(ids_hbm.at[pl.ds(c0 + base, IDXC)], idx_v)
                for j in range(NBUF):
                    pltpu.async_copy(x_hbm.at[c0 + base + j], bufs[j], sems[j])

                def body(g, carry2):
                    for j in range(NBUF):
                        i = g * NBUF + j
                        pltpu.make_async_copy(x_hbm.at[0], bufs[j], sems[j]).wait()
                        pltpu.sync_copy(bufs[j], acc.at[idx_v.at[i]], add=True)

                        @pl.when(i + NBUF < cnt_h)
                        def _():
                            pltpu.async_copy(
                                x_hbm.at[c0 + base + i + NBUF], bufs[j], sems[j]
                            )

                    return carry2

                lax.fori_loop(0, cnt_h // NBUF, body, 0)

            return carry

        lax.fori_loop(0, MAXC // IDXC, chunk_body, 0)
        plsc.subcore_barrier()

        for k in range(STRIPE // CHUNK):
            base = sid * STRIPE + k * CHUNK
            pltpu.sync_copy(acc.at[pl.ds(base, CHUNK)], out_hbm.at[cid, pl.ds(base, CHUNK)])

    return segsum(x3, ids2, zeros)


def _tc_matmul(partials, w):
    BS = 1000

    def mm(p_ref, w_ref, o_ref):
        p = p_ref[0] + p_ref[1]
        o_ref[...] = lax.dot_general(
            p, w_ref[...], (((1,), (1,)), ((), ())),
            preferred_element_type=jnp.float32,
        )

    return pl.pallas_call(
        mm,
        grid=(S // BS,),
        in_specs=[
            pl.BlockSpec((2, BS, D), lambda i: (0, i, 0)),
            pl.BlockSpec((M, D), lambda i: (0, 0)),
        ],
        out_specs=pl.BlockSpec((BS, M), lambda i: (i, 0)),
        out_shape=jax.ShapeDtypeStruct((S, M), jnp.float32),
    )(partials, w)


def kernel(tensor1_values, tensor1_segment_ids, tensor2_values):
    x3 = tensor1_values.reshape(NB, 128, D)
    ids2 = tensor1_segment_ids.astype(jnp.int32).reshape(NB, 128)
    ids2 = jnp.concatenate([ids2, jnp.zeros((NBP - NB, 128), jnp.int32)])
    zeros = jnp.zeros((128, D), jnp.float32)
    partials = _sc_segment_sum(x3, ids2, zeros)
    return _tc_matmul(partials, tensor2_values)
